# Initial kernel scaffold; baseline (speedup 1.0000x reference)
import jax
import jax.numpy as jnp
from jax.experimental import pallas as pl


def kernel(input, embedding):
    raise NotImplementedError("write your pallas kernel here")


# speedup vs baseline: 2.1018x; 2.1018x over previous
import functools

import jax
import jax.numpy as jnp
import numpy as np
from jax.experimental import pallas as pl
from jax.experimental.pallas import tpu as pltpu

NBAND = 1
NUM_CODE = 1024
CODE_DIM = 64
EPS = float(np.finfo(np.float32).eps)

TT = 512


def _vq_body(x_ref, emb_ref, q_ref, idx_ref, hist_ref, perp_ref, *, nb, nt, ntok):
    b = pl.program_id(0)
    t = pl.program_id(1)
    first = jnp.logical_and(b == 0, t == 0)
    last = jnp.logical_and(b == nb - 1, t == nt - 1)

    x = x_ref[0, 0]
    emb = emb_ref[0]

    e2 = jnp.sum(emb * emb, axis=1, keepdims=True)
    x2 = jnp.sum(x * x, axis=0, keepdims=True)
    d = jax.lax.dot_general(emb, x, (((1,), (0,)), ((), ())),
                            preferred_element_type=jnp.float32)
    dist = (x2 + e2) - 2.0 * d

    minv = jnp.min(dist, axis=0, keepdims=True)
    kio = jax.lax.broadcasted_iota(jnp.int32, dist.shape, 0)
    idx = jnp.min(jnp.where(dist == minv, kio, NUM_CODE), axis=0)
    idx_ref[0, 0] = idx

    oh = (kio == idx[None, :]).astype(jnp.float32)
    q = jax.lax.dot_general(emb, oh, (((0,), (0,)), ((), ())),
                            preferred_element_type=jnp.float32)
    q_ref[0, 0] = q

    @pl.when(first)
    def _():
        hist_ref[...] = jnp.zeros_like(hist_ref)

    hist_ref[...] += jnp.sum(oh, axis=1, keepdims=True)

    @pl.when(last)
    def _():
        p = hist_ref[...] * (1.0 / ntok)
        ent = jnp.sum(p * jnp.log(p + EPS))
        perp_ref[...] = jnp.full((1, 1), jnp.exp(-ent), dtype=jnp.float32)


@jax.jit
def kernel(input, embedding):
    B, C, N, T = input.shape
    nb, nt = B, T // TT
    ntok = B * T

    body = functools.partial(_vq_body, nb=nb, nt=nt, ntok=ntok)
    q, idx_raw, hist, perp = pl.pallas_call(
        body,
        grid=(nb, nt),
        in_specs=[
            pl.BlockSpec((1, 1, N, TT), lambda b, t: (b, 0, 0, t)),
            pl.BlockSpec((NBAND, NUM_CODE, CODE_DIM), lambda b, t: (0, 0, 0)),
        ],
        out_specs=[
            pl.BlockSpec((1, 1, N, TT), lambda b, t: (b, 0, 0, t)),
            pl.BlockSpec((1, 1, TT), lambda b, t: (b, 0, t)),
            pl.BlockSpec((NUM_CODE, 1), lambda b, t: (0, 0)),
            pl.BlockSpec((1, 1), lambda b, t: (0, 0)),
        ],
        out_shape=[
            jax.ShapeDtypeStruct((B, C, N, T), jnp.float32),
            jax.ShapeDtypeStruct((B, 1, T), jnp.int32),
            jax.ShapeDtypeStruct((NUM_CODE, 1), jnp.float32),
            jax.ShapeDtypeStruct((1, 1), jnp.float32),
        ],
        compiler_params=pltpu.CompilerParams(
            dimension_semantics=("arbitrary", "arbitrary"),
        ),
    )(input, embedding)

    return q, idx_raw.reshape(B, T, 1), perp.reshape(())


# speedup vs baseline: 2.4354x; 1.1587x over previous
import functools

import jax
import jax.numpy as jnp
import numpy as np
from jax.experimental import pallas as pl
from jax.experimental.pallas import tpu as pltpu

NBAND = 1
NUM_CODE = 1024
CODE_DIM = 64
EPS = float(np.finfo(np.float32).eps)

TT = 1024


def _vq_body(x_ref, emb_ref, q_ref, idx_ref, hist_ref, perp_ref, *, nb, nt, ntok):
    b = pl.program_id(0)
    t = pl.program_id(1)
    first = jnp.logical_and(b == 0, t == 0)
    last = jnp.logical_and(b == nb - 1, t == nt - 1)

    x = x_ref[0, 0]
    emb = emb_ref[0]

    e2 = jnp.sum(emb * emb, axis=1, keepdims=True)
    x2 = jnp.sum(x * x, axis=0, keepdims=True)
    d = jax.lax.dot_general(emb, x, (((1,), (0,)), ((), ())),
                            preferred_element_type=jnp.float32)
    dist = (x2 + e2) - 2.0 * d

    minv = jnp.min(dist, axis=0, keepdims=True)
    kio = jax.lax.broadcasted_iota(jnp.int32, dist.shape, 0).astype(jnp.float32)
    idx_f = jnp.min(jnp.where(dist == minv, kio, float(NUM_CODE)),
                    axis=0)
    idx_ref[0, 0] = idx_f.astype(jnp.int32)

    oh = (kio == idx_f[None, :]).astype(jnp.float32)
    q = jax.lax.dot_general(emb, oh, (((0,), (0,)), ((), ())),
                            preferred_element_type=jnp.float32)
    q_ref[0, 0] = q

    @pl.when(first)
    def _():
        hist_ref[...] = jnp.zeros_like(hist_ref)

    ones = jnp.ones((oh.shape[1], 1), jnp.float32)
    hist_ref[...] += jax.lax.dot_general(oh, ones, (((1,), (0,)), ((), ())),
                                         preferred_element_type=jnp.float32)

    @pl.when(last)
    def _():
        p = hist_ref[...] * (1.0 / ntok)
        ent = jnp.sum(p * jnp.log(p + EPS))
        perp_ref[...] = jnp.full((1, 1), jnp.exp(-ent), dtype=jnp.float32)


@jax.jit
def kernel(input, embedding):
    B, C, N, T = input.shape
    nb, nt = B, T // TT
    ntok = B * T

    body = functools.partial(_vq_body, nb=nb, nt=nt, ntok=ntok)
    q, idx_raw, hist, perp = pl.pallas_call(
        body,
        grid=(nb, nt),
        in_specs=[
            pl.BlockSpec((1, 1, N, TT), lambda b, t: (b, 0, 0, t)),
            pl.BlockSpec((NBAND, NUM_CODE, CODE_DIM), lambda b, t: (0, 0, 0)),
        ],
        out_specs=[
            pl.BlockSpec((1, 1, N, TT), lambda b, t: (b, 0, 0, t)),
            pl.BlockSpec((1, 1, TT), lambda b, t: (b, 0, t)),
            pl.BlockSpec((NUM_CODE, 1), lambda b, t: (0, 0)),
            pl.BlockSpec((1, 1), lambda b, t: (0, 0)),
        ],
        out_shape=[
            jax.ShapeDtypeStruct((B, C, N, T), jnp.float32),
            jax.ShapeDtypeStruct((B, 1, T), jnp.int32),
            jax.ShapeDtypeStruct((NUM_CODE, 1), jnp.float32),
            jax.ShapeDtypeStruct((1, 1), jnp.float32),
        ],
        compiler_params=pltpu.CompilerParams(
            dimension_semantics=("arbitrary", "arbitrary"),
        ),
    )(input, embedding)

    return q, idx_raw.reshape(B, T, 1), perp.reshape(())


# speedup vs baseline: 2.6163x; 1.0743x over previous
import functools

import jax
import jax.numpy as jnp
import numpy as np
from jax.experimental import pallas as pl
from jax.experimental.pallas import tpu as pltpu

NBAND = 1
NUM_CODE = 1024
CODE_DIM = 64
EPS = float(np.finfo(np.float32).eps)

TT = 1024


def _vq_body(x_ref, emb_ref, q_ref, idx_ref, perp_ref, acc_ref, *, nb, nt, ntok):
    b = pl.program_id(0)
    t = pl.program_id(1)
    first = jnp.logical_and(b == 0, t == 0)
    last = jnp.logical_and(b == nb - 1, t == nt - 1)

    x = x_ref[0, 0]
    emb = emb_ref[0]

    e2 = jnp.sum(emb * emb, axis=1, keepdims=True)
    x2 = jnp.sum(x * x, axis=0, keepdims=True)
    d = jax.lax.dot_general(emb, x, (((1,), (0,)), ((), ())),
                            preferred_element_type=jnp.float32)
    dist = (x2 + e2) - 2.0 * d

    minv = jnp.min(dist, axis=0, keepdims=True)
    kio = jax.lax.broadcasted_iota(jnp.int32, dist.shape, 0).astype(jnp.float32)
    idx_f = jnp.min(jnp.where(dist == minv, kio, float(NUM_CODE)),
                    axis=0)
    idx_ref[0, 0] = idx_f.astype(jnp.int32)

    oh = (kio == idx_f[None, :]).astype(jnp.float32)
    q = jax.lax.dot_general(emb, oh, (((0,), (0,)), ((), ())),
                            preferred_element_type=jnp.float32)
    q_ref[0, 0] = q

    @pl.when(first)
    def _():
        acc_ref[...] = oh

    @pl.when(jnp.logical_not(first))
    def _():
        acc_ref[...] += oh

    @pl.when(last)
    def _():
        hist = jnp.sum(acc_ref[...], axis=1, keepdims=True)
        p = hist * (1.0 / ntok)
        ent = jnp.sum(p * jnp.log(p + EPS))
        perp_ref[...] = jnp.full((1, 1), jnp.exp(-ent), dtype=jnp.float32)


@jax.jit
def kernel(input, embedding):
    B, C, N, T = input.shape
    nb, nt = B, T // TT
    ntok = B * T

    body = functools.partial(_vq_body, nb=nb, nt=nt, ntok=ntok)
    q, idx_raw, perp = pl.pallas_call(
        body,
        grid=(nb, nt),
        in_specs=[
            pl.BlockSpec((1, 1, N, TT), lambda b, t: (b, 0, 0, t)),
            pl.BlockSpec((NBAND, NUM_CODE, CODE_DIM), lambda b, t: (0, 0, 0)),
        ],
        out_specs=[
            pl.BlockSpec((1, 1, N, TT), lambda b, t: (b, 0, 0, t)),
            pl.BlockSpec((1, 1, TT), lambda b, t: (b, 0, t)),
            pl.BlockSpec((1, 1), lambda b, t: (0, 0)),
        ],
        out_shape=[
            jax.ShapeDtypeStruct((B, C, N, T), jnp.float32),
            jax.ShapeDtypeStruct((B, 1, T), jnp.int32),
            jax.ShapeDtypeStruct((1, 1), jnp.float32),
        ],
        scratch_shapes=[pltpu.VMEM((NUM_CODE, TT), jnp.float32)],
        compiler_params=pltpu.CompilerParams(
            dimension_semantics=("arbitrary", "arbitrary"),
        ),
    )(input, embedding)

    return q, idx_raw.reshape(B, T, 1), perp.reshape(())


# speedup vs baseline: 2.6472x; 1.0118x over previous
import functools

import jax
import jax.numpy as jnp
import numpy as np
from jax.experimental import pallas as pl
from jax.experimental.pallas import tpu as pltpu

NBAND = 1
NUM_CODE = 1024
CODE_DIM = 64
EPS = float(np.finfo(np.float32).eps)

TT = 1024


def _vq_body(x_ref, emb_ref, q_ref, idx_ref, perp_ref, acc_ref, *, nb, nt, ntok):
    b = pl.program_id(0)
    t = pl.program_id(1)
    first = jnp.logical_and(b == 0, t == 0)
    last = jnp.logical_and(b == nb - 1, t == nt - 1)

    x = x_ref[0, 0]
    emb = emb_ref[0]

    e2 = jnp.sum(emb * emb, axis=1, keepdims=True)
    x2 = jnp.sum(x * x, axis=0, keepdims=True)
    d = jax.lax.dot_general(emb, x, (((1,), (0,)), ((), ())),
                            preferred_element_type=jnp.float32)
    dist = (x2 * 0.5 + e2 * 0.5) - d

    minv = jnp.min(dist, axis=0, keepdims=True)
    kio = jax.lax.broadcasted_iota(jnp.int32, dist.shape, 0).astype(jnp.float32)
    idx_f = jnp.min(jnp.where(dist == minv, kio, float(NUM_CODE)),
                    axis=0)
    idx_ref[0, 0] = idx_f.astype(jnp.int32)

    oh = (kio == idx_f[None, :]).astype(jnp.float32)
    q = jax.lax.dot_general(emb, oh, (((0,), (0,)), ((), ())),
                            preferred_element_type=jnp.float32)
    q_ref[0, 0] = q

    @pl.when(first)
    def _():
        acc_ref[...] = oh

    @pl.when(jnp.logical_not(first))
    def _():
        acc_ref[...] += oh

    @pl.when(last)
    def _():
        hist = jnp.sum(acc_ref[...], axis=1, keepdims=True)
        p = hist * (1.0 / ntok)
        ent = jnp.sum(p * jnp.log(p + EPS))
        perp_ref[...] = jnp.full((1, 1), jnp.exp(-ent), dtype=jnp.float32)


@jax.jit
def kernel(input, embedding):
    B, C, N, T = input.shape
    nb, nt = B, T // TT
    ntok = B * T

    body = functools.partial(_vq_body, nb=nb, nt=nt, ntok=ntok)
    q, idx_raw, perp = pl.pallas_call(
        body,
        grid=(nb, nt),
        in_specs=[
            pl.BlockSpec((1, 1, N, TT), lambda b, t: (b, 0, 0, t)),
            pl.BlockSpec((NBAND, NUM_CODE, CODE_DIM), lambda b, t: (0, 0, 0)),
        ],
        out_specs=[
            pl.BlockSpec((1, 1, N, TT), lambda b, t: (b, 0, 0, t)),
            pl.BlockSpec((1, 1, TT), lambda b, t: (b, 0, t)),
            pl.BlockSpec((1, 1), lambda b, t: (0, 0)),
        ],
        out_shape=[
            jax.ShapeDtypeStruct((B, C, N, T), jnp.float32),
            jax.ShapeDtypeStruct((B, 1, T), jnp.int32),
            jax.ShapeDtypeStruct((1, 1), jnp.float32),
        ],
        scratch_shapes=[pltpu.VMEM((NUM_CODE, TT), jnp.float32)],
        compiler_params=pltpu.CompilerParams(
            dimension_semantics=("arbitrary", "arbitrary"),
        ),
    )(input, embedding)

    return q, idx_raw.reshape(B, T, 1), perp.reshape(())


# speedup vs baseline: 2.8243x; 1.0669x over previous
import functools

import jax
import jax.numpy as jnp
import numpy as np
from jax.experimental import pallas as pl
from jax.experimental.pallas import tpu as pltpu

NBAND = 1
NUM_CODE = 1024
CODE_DIM = 64
EPS = float(np.finfo(np.float32).eps)

TT = 2048


def _vq_body(x_ref, emb_ref, q_ref, idx_ref, perp_ref, acc_ref, *, nb, nt, ntok):
    b = pl.program_id(0)
    t = pl.program_id(1)
    first = jnp.logical_and(b == 0, t == 0)
    last = jnp.logical_and(b == nb - 1, t == nt - 1)

    x = x_ref[0, 0]
    emb = emb_ref[0]

    e2 = jnp.sum(emb * emb, axis=1, keepdims=True)
    x2 = jnp.sum(x * x, axis=0, keepdims=True)
    d = jax.lax.dot_general(emb, x, (((1,), (0,)), ((), ())),
                            preferred_element_type=jnp.float32)
    dist = (x2 * 0.5 + e2 * 0.5) - d

    minv = jnp.min(dist, axis=0, keepdims=True)
    kio = jax.lax.broadcasted_iota(jnp.int32, dist.shape, 0).astype(jnp.float32)
    idx_f = jnp.min(jnp.where(dist == minv, kio, float(NUM_CODE)),
                    axis=0)
    idx_ref[0, 0] = idx_f.astype(jnp.int32)

    oh = (kio == idx_f[None, :]).astype(jnp.float32)
    q = jax.lax.dot_general(emb, oh, (((0,), (0,)), ((), ())),
                            preferred_element_type=jnp.float32)
    q_ref[0, 0] = q

    @pl.when(first)
    def _():
        acc_ref[...] = oh

    @pl.when(jnp.logical_not(first))
    def _():
        acc_ref[...] += oh

    @pl.when(last)
    def _():
        hist = jnp.sum(acc_ref[...], axis=1, keepdims=True)
        p = hist * (1.0 / ntok)
        ent = jnp.sum(p * jnp.log(p + EPS))
        perp_ref[...] = jnp.full((1, 1), jnp.exp(-ent), dtype=jnp.float32)


@jax.jit
def kernel(input, embedding):
    B, C, N, T = input.shape
    nb, nt = B, T // TT
    ntok = B * T

    body = functools.partial(_vq_body, nb=nb, nt=nt, ntok=ntok)
    q, idx_raw, perp = pl.pallas_call(
        body,
        grid=(nb, nt),
        in_specs=[
            pl.BlockSpec((1, 1, N, TT), lambda b, t: (b, 0, 0, t)),
            pl.BlockSpec((NBAND, NUM_CODE, CODE_DIM), lambda b, t: (0, 0, 0)),
        ],
        out_specs=[
            pl.BlockSpec((1, 1, N, TT), lambda b, t: (b, 0, 0, t)),
            pl.BlockSpec((1, 1, TT), lambda b, t: (b, 0, t)),
            pl.BlockSpec((1, 1), lambda b, t: (0, 0)),
        ],
        out_shape=[
            jax.ShapeDtypeStruct((B, C, N, T), jnp.float32),
            jax.ShapeDtypeStruct((B, 1, T), jnp.int32),
            jax.ShapeDtypeStruct((1, 1), jnp.float32),
        ],
        scratch_shapes=[pltpu.VMEM((NUM_CODE, TT), jnp.float32)],
        compiler_params=pltpu.CompilerParams(
            dimension_semantics=("arbitrary", "arbitrary"),
        ),
    )(input, embedding)

    return q, idx_raw.reshape(B, T, 1), perp.reshape(())


# speedup vs baseline: 3.7759x; 1.3369x over previous
import functools

import jax
import jax.numpy as jnp
import numpy as np
from jax.experimental import pallas as pl
from jax.experimental.pallas import tpu as pltpu

NBAND = 1
NUM_CODE = 1024
CODE_DIM = 64
EPS = float(np.finfo(np.float32).eps)

TT = 2048


def _vq_body(x_ref, emb_ref, q_ref, idx_ref, perp_ref, acc_ref, *, nb, nt, ntok):
    b = pl.program_id(0)
    t = pl.program_id(1)
    first = jnp.logical_and(b == 0, t == 0)
    last = jnp.logical_and(b == nb - 1, t == nt - 1)

    x = x_ref[0, 0]
    emb = emb_ref[0]

    e2 = jnp.sum(emb * emb, axis=1, keepdims=True)
    x2 = jnp.sum(x * x, axis=0, keepdims=True)
    d = jax.lax.dot_general(emb, x, (((1,), (0,)), ((), ())),
                            preferred_element_type=jnp.float32)
    dist = (x2 * 0.5 + e2 * 0.5) - d

    idx_i = jnp.argmin(dist, axis=0)
    idx_ref[0, 0] = idx_i
    kio_i = jax.lax.broadcasted_iota(jnp.int32, dist.shape, 0)
    mask = kio_i == idx_i[None, :]
    oh = mask.astype(jnp.float32)
    q = jax.lax.dot_general(emb, oh, (((0,), (0,)), ((), ())),
                            preferred_element_type=jnp.float32)
    q_ref[0, 0] = q

    cnt = jnp.sum(mask, axis=1, keepdims=True)

    @pl.when(first)
    def _():
        acc_ref[...] = cnt

    @pl.when(jnp.logical_not(first))
    def _():
        acc_ref[...] += cnt

    @pl.when(last)
    def _():
        p = acc_ref[...].astype(jnp.float32) * (1.0 / ntok)
        ent = jnp.sum(p * jnp.log(p + EPS))
        perp_ref[...] = jnp.full((1, 1), jnp.exp(-ent), dtype=jnp.float32)


@jax.jit
def kernel(input, embedding):
    B, C, N, T = input.shape
    nb, nt = B, T // TT
    ntok = B * T

    body = functools.partial(_vq_body, nb=nb, nt=nt, ntok=ntok)
    q, idx_raw, perp = pl.pallas_call(
        body,
        grid=(nb, nt),
        in_specs=[
            pl.BlockSpec((1, 1, N, TT), lambda b, t: (b, 0, 0, t)),
            pl.BlockSpec((NBAND, NUM_CODE, CODE_DIM), lambda b, t: (0, 0, 0)),
        ],
        out_specs=[
            pl.BlockSpec((1, 1, N, TT), lambda b, t: (b, 0, 0, t)),
            pl.BlockSpec((1, 1, TT), lambda b, t: (b, 0, t)),
            pl.BlockSpec((1, 1), lambda b, t: (0, 0)),
        ],
        out_shape=[
            jax.ShapeDtypeStruct((B, C, N, T), jnp.float32),
            jax.ShapeDtypeStruct((B, 1, T), jnp.int32),
            jax.ShapeDtypeStruct((1, 1), jnp.float32),
        ],
        scratch_shapes=[pltpu.VMEM((NUM_CODE, 1), jnp.int32)],
        compiler_params=pltpu.CompilerParams(
            dimension_semantics=("arbitrary", "arbitrary"),
        ),
    )(input, embedding)

    return q, idx_raw.reshape(B, T, 1), perp.reshape(())


# speedup vs baseline: 3.9824x; 1.0547x over previous
import functools

import jax
import jax.numpy as jnp
import numpy as np
from jax.experimental import pallas as pl
from jax.experimental.pallas import tpu as pltpu

NBAND = 1
NUM_CODE = 1024
CODE_DIM = 64
EPS = float(np.finfo(np.float32).eps)

TT = 2048
BB = 2


def _vq_body(x_ref, emb_ref, q_ref, idx_ref, perp_ref, acc_ref, *, ng, ntok):
    g = pl.program_id(0)
    first = g == 0
    last = g == ng - 1

    emb = emb_ref[0]
    e2 = jnp.sum(emb * emb, axis=1, keepdims=True)

    cnt = acc_ref[...]
    cnt = jnp.where(first, jnp.zeros_like(cnt), cnt)

    for j in range(BB):
        x = x_ref[j, 0]
        x2 = jnp.sum(x * x, axis=0, keepdims=True)
        d = jax.lax.dot_general(emb, x, (((1,), (0,)), ((), ())),
                                preferred_element_type=jnp.float32)
        dist = (x2 * 0.5 + e2 * 0.5) - d

        idx_i = jnp.argmin(dist, axis=0)
        idx_ref[j, 0] = idx_i
        kio_i = jax.lax.broadcasted_iota(jnp.int32, dist.shape, 0)
        oh = (kio_i == idx_i[None, :]).astype(jnp.float32)
        q = jax.lax.dot_general(emb, oh, (((0,), (0,)), ((), ())),
                                preferred_element_type=jnp.float32)
        q_ref[j, 0] = q

        cnt = cnt + jnp.sum(oh, axis=1, keepdims=True)

    acc_ref[...] = cnt

    @pl.when(last)
    def _():
        p = cnt * (1.0 / ntok)
        ent = jnp.sum(p * jnp.log(p + EPS))
        perp_ref[...] = jnp.full((1, 1), jnp.exp(-ent), dtype=jnp.float32)


@jax.jit
def kernel(input, embedding):
    B, C, N, T = input.shape
    ng = B // BB
    ntok = B * T

    body = functools.partial(_vq_body, ng=ng, ntok=ntok)
    q, idx_raw, perp = pl.pallas_call(
        body,
        grid=(ng,),
        in_specs=[
            pl.BlockSpec((BB, 1, N, TT), lambda g: (g, 0, 0, 0)),
            pl.BlockSpec((NBAND, NUM_CODE, CODE_DIM), lambda g: (0, 0, 0)),
        ],
        out_specs=[
            pl.BlockSpec((BB, 1, N, TT), lambda g: (g, 0, 0, 0)),
            pl.BlockSpec((BB, 1, TT), lambda g: (g, 0, 0)),
            pl.BlockSpec((1, 1), lambda g: (0, 0)),
        ],
        out_shape=[
            jax.ShapeDtypeStruct((B, C, N, T), jnp.float32),
            jax.ShapeDtypeStruct((B, 1, T), jnp.int32),
            jax.ShapeDtypeStruct((1, 1), jnp.float32),
        ],
        scratch_shapes=[pltpu.VMEM((NUM_CODE, 1), jnp.float32)],
        compiler_params=pltpu.CompilerParams(
            dimension_semantics=("arbitrary",),
        ),
    )(input, embedding)

    return q, idx_raw.reshape(B, T, 1), perp.reshape(())


# speedup vs baseline: 3.9942x; 1.0030x over previous
import functools

import jax
import jax.numpy as jnp
import numpy as np
from jax.experimental import pallas as pl
from jax.experimental.pallas import tpu as pltpu

NBAND = 1
NUM_CODE = 1024
CODE_DIM = 64
EPS = float(np.finfo(np.float32).eps)

TT = 2048
BB = 4


def _vq_body(x_ref, emb_ref, q_ref, idx_ref, perp_ref, acc_ref, *, ng, ntok):
    g = pl.program_id(0)
    first = g == 0
    last = g == ng - 1

    emb = emb_ref[0]
    e2 = jnp.sum(emb * emb, axis=1, keepdims=True)

    cnt = acc_ref[...]
    cnt = jnp.where(first, jnp.zeros_like(cnt), cnt)

    for j in range(BB):
        x = x_ref[j, 0]
        x2 = jnp.sum(x * x, axis=0, keepdims=True)
        d = jax.lax.dot_general(emb, x, (((1,), (0,)), ((), ())),
                                preferred_element_type=jnp.float32)
        dist = (x2 * 0.5 + e2 * 0.5) - d

        idx_i = jnp.argmin(dist, axis=0)
        idx_ref[j, 0] = idx_i
        kio_i = jax.lax.broadcasted_iota(jnp.int32, dist.shape, 0)
        oh = (kio_i == idx_i[None, :]).astype(jnp.float32)
        q = jax.lax.dot_general(emb, oh, (((0,), (0,)), ((), ())),
                                preferred_element_type=jnp.float32)
        q_ref[j, 0] = q

        cnt = cnt + jnp.sum(oh, axis=1, keepdims=True)

    acc_ref[...] = cnt

    @pl.when(last)
    def _():
        p = cnt * (1.0 / ntok)
        ent = jnp.sum(p * jnp.log(p + EPS))
        perp_ref[...] = jnp.full((1, 1), jnp.exp(-ent), dtype=jnp.float32)


@jax.jit
def kernel(input, embedding):
    B, C, N, T = input.shape
    ng = B // BB
    ntok = B * T

    body = functools.partial(_vq_body, ng=ng, ntok=ntok)
    q, idx_raw, perp = pl.pallas_call(
        body,
        grid=(ng,),
        in_specs=[
            pl.BlockSpec((BB, 1, N, TT), lambda g: (g, 0, 0, 0)),
            pl.BlockSpec((NBAND, NUM_CODE, CODE_DIM), lambda g: (0, 0, 0)),
        ],
        out_specs=[
            pl.BlockSpec((BB, 1, N, TT), lambda g: (g, 0, 0, 0)),
            pl.BlockSpec((BB, 1, TT), lambda g: (g, 0, 0)),
            pl.BlockSpec((1, 1), lambda g: (0, 0)),
        ],
        out_shape=[
            jax.ShapeDtypeStruct((B, C, N, T), jnp.float32),
            jax.ShapeDtypeStruct((B, 1, T), jnp.int32),
            jax.ShapeDtypeStruct((1, 1), jnp.float32),
        ],
        scratch_shapes=[pltpu.VMEM((NUM_CODE, 1), jnp.float32)],
        compiler_params=pltpu.CompilerParams(
            dimension_semantics=("arbitrary",),
        ),
    )(input, embedding)

    return q, idx_raw.reshape(B, T, 1), perp.reshape(())
